# baseline (speedup 1.0000x reference)
import functools

import jax
import jax.numpy as jnp
from jax import lax
from jax.experimental import pallas as pl
from jax.experimental.pallas import tpu as pltpu
from jax.experimental.pallas import tpu_sc as plsc

VOCAB = 100000
HIDDEN = 1024
MAXPOS = 2048
TYPES = 2
EPS = 1e-12
BATCH = 4
SEQ = 2048

N_TOKENS = BATCH * SEQ

NC = 2
NS = 16
NW = NC * NS

N_SLICES = 8
SEQ_SLICE = SEQ // N_SLICES
SLICE = BATCH * SEQ_SLICE
TOK_PER_W = SLICE // NW
CHUNK = 32
N_CHUNKS = TOK_PER_W // CHUNK


def _sc_gather(s, ids_flat, token_table):
    mesh = plsc.VectorSubcoreMesh(core_axis_name="c", subcore_axis_name="s")

    @functools.partial(
        pl.kernel,
        mesh=mesh,
        out_type=jax.ShapeDtypeStruct((SLICE, HIDDEN), jnp.float32),
        scratch_types=[
            pltpu.VMEM((TOK_PER_W,), jnp.int32),
            pltpu.VMEM((N_CHUNKS, CHUNK, HIDDEN), jnp.float32),
            pltpu.SemaphoreType.DMA((N_CHUNKS,)),
            pltpu.SemaphoreType.DMA((N_CHUNKS,)),
        ],
    )
    def k(table_hbm, ids_hbm, out_hbm, idx_v, rows_v, gsem, osem):
        wid = lax.axis_index("s") * NC + lax.axis_index("c")
        r = wid * TOK_PER_W
        b = r // SEQ_SLICE
        p0 = r % SEQ_SLICE
        gbase = b * SEQ + s * SEQ_SLICE + p0
        pltpu.sync_copy(ids_hbm.at[pl.ds(gbase, TOK_PER_W)], idx_v)
        gathers = []
        for c in range(N_CHUNKS):
            g = pltpu.async_copy(
                table_hbm.at[idx_v.at[pl.ds(c * CHUNK, CHUNK)]],
                rows_v.at[c], gsem.at[c])
            gathers.append(g)
        outs = []
        for c in range(N_CHUNKS):
            gathers[c].wait()
            o = pltpu.async_copy(
                rows_v.at[c], out_hbm.at[pl.ds(r + c * CHUNK, CHUNK)],
                osem.at[c])
            outs.append(o)
        for o in outs:
            o.wait()

    return k(token_table, ids_flat)


ROWS_BLK = 256
BLK_PER_SEQ_SLICE = SEQ_SLICE // ROWS_BLK
SEQ_BLOCKS = SEQ // ROWS_BLK


def _tc_epilogue_body(tid_ref, tok_ref, pos_ref, typ_ref, gamma_ref, beta_ref,
                      *rest):
    out_ref = rest[-1]
    x = tok_ref[...] + pos_ref[...]
    te = jnp.where(tid_ref[...] == 0, typ_ref[0:1, :], typ_ref[1:2, :])
    x = x + te
    mean = jnp.mean(x, axis=-1, keepdims=True)
    xc = x - mean
    var = jnp.mean(xc * xc, axis=-1, keepdims=True)
    y = xc * lax.rsqrt(var + EPS)
    out_ref[...] = y * gamma_ref[...] + beta_ref[...]


def _tc_epilogue_slice(s, acc, tok_rows, tid_col, pos_table, type_table,
                       gamma2d, beta2d):
    blk0 = s * BLK_PER_SEQ_SLICE

    in_specs = [
        pl.BlockSpec((ROWS_BLK, 1),
                     lambda i, b: (b * SEQ_BLOCKS + blk0 + i, 0)),
        pl.BlockSpec((ROWS_BLK, HIDDEN),
                     lambda i, b: (b * BLK_PER_SEQ_SLICE + i, 0)),
        pl.BlockSpec((ROWS_BLK, HIDDEN), lambda i, b: (blk0 + i, 0)),
        pl.BlockSpec((TYPES, HIDDEN), lambda i, b: (0, 0)),
        pl.BlockSpec((1, HIDDEN), lambda i, b: (0, 0)),
        pl.BlockSpec((1, HIDDEN), lambda i, b: (0, 0)),
    ]
    args = [tid_col, tok_rows, pos_table, type_table, gamma2d, beta2d]
    io_aliases = {}
    if acc is not None:
        in_specs.append(pl.BlockSpec(memory_space=pl.ANY))
        args.append(acc)
        io_aliases = {6: 0}
    return pl.pallas_call(
        _tc_epilogue_body,
        grid=(BLK_PER_SEQ_SLICE, BATCH),
        in_specs=in_specs,
        out_specs=pl.BlockSpec(
            (ROWS_BLK, HIDDEN),
            lambda i, b: (b * SEQ_BLOCKS + blk0 + i, 0)),
        out_shape=jax.ShapeDtypeStruct((N_TOKENS, HIDDEN), jnp.float32),
        input_output_aliases=io_aliases,
    )(*args)


def kernel(input_ids, token_type_ids, token_table, pos_table, type_table,
           gamma, beta):
    ids_flat = input_ids.reshape(-1).astype(jnp.int32)
    tid_col = token_type_ids.reshape(N_TOKENS, 1).astype(jnp.int32)
    gamma2d = gamma.reshape(1, HIDDEN)
    beta2d = beta.reshape(1, HIDDEN)

    gathered = [_sc_gather(s, ids_flat, token_table)
                for s in range(N_SLICES)]
    acc = None
    for s in range(N_SLICES):
        acc = _tc_epilogue_slice(s, acc, gathered[s], tid_col,
                                 pos_table, type_table, gamma2d, beta2d)
    return acc.reshape(BATCH, SEQ, HIDDEN)


# speedup vs baseline: 1.1786x; 1.1786x over previous
import functools

import jax
import jax.numpy as jnp
from jax import lax
from jax.experimental import pallas as pl
from jax.experimental.pallas import tpu as pltpu
from jax.experimental.pallas import tpu_sc as plsc

VOCAB = 100000
HIDDEN = 1024
MAXPOS = 2048
TYPES = 2
EPS = 1e-12
BATCH = 4
SEQ = 2048

N_TOKENS = BATCH * SEQ

NC = 2
NS = 16
NW = NC * NS

N_SLICES = 4
SEQ_SLICE = SEQ // N_SLICES
SLICE = BATCH * SEQ_SLICE
TOK_PER_W = SLICE // NW
CHUNK = 32
N_CHUNKS = TOK_PER_W // CHUNK


def _sc_gather(s, ids_flat, token_table):
    mesh = plsc.VectorSubcoreMesh(core_axis_name="c", subcore_axis_name="s")

    @functools.partial(
        pl.kernel,
        mesh=mesh,
        out_type=jax.ShapeDtypeStruct((SLICE, HIDDEN), jnp.float32),
        scratch_types=[
            pltpu.VMEM((TOK_PER_W,), jnp.int32),
            pltpu.VMEM((N_CHUNKS, CHUNK, HIDDEN), jnp.float32),
            pltpu.SemaphoreType.DMA((N_CHUNKS,)),
            pltpu.SemaphoreType.DMA((N_CHUNKS,)),
        ],
    )
    def k(table_hbm, ids_hbm, out_hbm, idx_v, rows_v, gsem, osem):
        wid = lax.axis_index("s") * NC + lax.axis_index("c")
        r = wid * TOK_PER_W
        b = r // SEQ_SLICE
        p0 = r % SEQ_SLICE
        gbase = b * SEQ + s * SEQ_SLICE + p0
        pltpu.sync_copy(ids_hbm.at[pl.ds(gbase, TOK_PER_W)], idx_v)
        gathers = []
        for c in range(N_CHUNKS):
            g = pltpu.async_copy(
                table_hbm.at[idx_v.at[pl.ds(c * CHUNK, CHUNK)]],
                rows_v.at[c], gsem.at[c])
            gathers.append(g)
        outs = []
        for c in range(N_CHUNKS):
            gathers[c].wait()
            o = pltpu.async_copy(
                rows_v.at[c], out_hbm.at[pl.ds(r + c * CHUNK, CHUNK)],
                osem.at[c])
            outs.append(o)
        for o in outs:
            o.wait()

    return k(token_table, ids_flat)


ROWS_BLK = 256
HALF_PER_SLICE = SEQ_SLICE // ROWS_BLK
OUT_BLOCKS = N_TOKENS // ROWS_BLK


def _ln_one(tid_ref, tok_ref, pos_ref, typ_ref, gamma_ref, beta_ref, out_ref):
    x = tok_ref[...] + pos_ref[...]
    te = jnp.where(tid_ref[...] == 0, typ_ref[0:1, :], typ_ref[1:2, :])
    x = x + te
    mean = jnp.mean(x, axis=-1, keepdims=True)
    xc = x - mean
    var = jnp.mean(xc * xc, axis=-1, keepdims=True)
    y = xc * lax.rsqrt(var + EPS)
    out_ref[...] = y * gamma_ref[...] + beta_ref[...]


class _HalfView:
    def __init__(self, ref, h):
        self._ref = ref
        self._h = h

    def __getitem__(self, idx):
        raise NotImplementedError

    def __setitem__(self, idx, val):
        self._ref[pl.ds(self._h * ROWS_BLK, ROWS_BLK), :] = val


def _tc_epilogue_body(tid_a, tid_b, tok_a, tok_b, pos_a, pos_b, typ_ref,
                      gamma_ref, beta_ref, *rest):
    out_ref = rest[-1]
    _ln_one(tid_a, tok_a, pos_a, typ_ref, gamma_ref, beta_ref,
            _HalfView(out_ref, 0))
    _ln_one(tid_b, tok_b, pos_b, typ_ref, gamma_ref, beta_ref,
            _HalfView(out_ref, 1))


def _tc_epilogue_slice(s, acc, tok_rows, tid_col, pos_table, type_table,
                       gamma2d, beta2d):
    blk0 = s * HALF_PER_SLICE

    def tid_map(h):
        return lambda b: (b * (SEQ // ROWS_BLK) + blk0 + h, 0)

    def tok_map(h):
        return lambda b: (b * HALF_PER_SLICE + h, 0)

    def pos_map(h):
        return lambda b: (blk0 + h, 0)

    in_specs = [
        pl.BlockSpec((ROWS_BLK, 1), tid_map(0)),
        pl.BlockSpec((ROWS_BLK, 1), tid_map(1)),
        pl.BlockSpec((ROWS_BLK, HIDDEN), tok_map(0)),
        pl.BlockSpec((ROWS_BLK, HIDDEN), tok_map(1)),
        pl.BlockSpec((ROWS_BLK, HIDDEN), pos_map(0)),
        pl.BlockSpec((ROWS_BLK, HIDDEN), pos_map(1)),
        pl.BlockSpec((TYPES, HIDDEN), lambda b: (0, 0)),
        pl.BlockSpec((1, HIDDEN), lambda b: (0, 0)),
        pl.BlockSpec((1, HIDDEN), lambda b: (0, 0)),
    ]
    args = [tid_col, tid_col, tok_rows, tok_rows, pos_table, pos_table,
            type_table, gamma2d, beta2d]
    io_aliases = {}
    if acc is not None:
        in_specs.append(pl.BlockSpec(memory_space=pl.ANY))
        args.append(acc)
        io_aliases = {9: 0}
    return pl.pallas_call(
        _tc_epilogue_body,
        grid=(BATCH,),
        in_specs=in_specs,
        out_specs=pl.BlockSpec((2 * ROWS_BLK, HIDDEN),
                               lambda b: (b * N_SLICES + s, 0)),
        out_shape=jax.ShapeDtypeStruct((N_TOKENS, HIDDEN), jnp.float32),
        input_output_aliases=io_aliases,
    )(*args)


def kernel(input_ids, token_type_ids, token_table, pos_table, type_table,
           gamma, beta):
    ids_flat = input_ids.reshape(-1).astype(jnp.int32)
    tid_col = token_type_ids.reshape(N_TOKENS, 1).astype(jnp.int32)
    gamma2d = gamma.reshape(1, HIDDEN)
    beta2d = beta.reshape(1, HIDDEN)

    gathered = [_sc_gather(s, ids_flat, token_table)
                for s in range(N_SLICES)]
    acc = None
    for s in range(N_SLICES):
        acc = _tc_epilogue_slice(s, acc, gathered[s], tid_col,
                                 pos_table, type_table, gamma2d, beta2d)
    return acc.reshape(BATCH, SEQ, HIDDEN)
